# Initial kernel scaffold; baseline (speedup 1.0000x reference)
import jax
import jax.numpy as jnp
from jax.experimental import pallas as pl


def kernel(x, edge_index, batch, target, params):
    raise NotImplementedError("write your pallas kernel here")


# speedup vs baseline: 4.8036x; 4.8036x over previous
import functools

import jax
import jax.numpy as jnp
from jax import lax
from jax.experimental import pallas as pl
from jax.experimental.pallas import tpu as pltpu
from jax.experimental.pallas import tpu_sc as plsc

_NN = 50000
_NWRK = 32
_NPW = 1568
_NNP = _NWRK * _NPW
_NE = 800000
_CH = 128
_NEP = _NE + _CH
_NG = 256
_BM = 1024
_GRID_M = _NNP // _BM


def _spmm_body(w, h_hbm, srcs_hbm, dsts_hbm, nrms_hbm, rs_hbm, zeros_hbm,
               o_hbm, rs_v, src_v, dst_v, nrm_v, rows_v, acc_v,
               s0, s1, s2, s3):
    grp = w // 16
    wid = lax.axis_index("s") * 2 + lax.axis_index("c")
    nbase = wid * _NPW
    pltpu.sync_copy(rs_hbm, rs_v)
    pltpu.sync_copy(zeros_hbm, acc_v)
    e0 = rs_v[pl.ds(wid, 16)][0]
    e1 = rs_v[pl.ds(wid + 1, 16)][0]
    a0 = (e0 // 8) * 8
    nch = (e1 - a0 + _CH - 1) // _CH

    def chunk_body(k, _):
        base = a0 + k * _CH
        d0 = pltpu.async_copy(srcs_hbm.at[pl.ds(base, _CH)], src_v, s0)
        d1 = pltpu.async_copy(dsts_hbm.at[pl.ds(base, _CH)],
                              dst_v.at[pl.ds(0, _CH)], s1)
        d2 = pltpu.async_copy(nrms_hbm.at[pl.ds(base, _CH)],
                              nrm_v.at[pl.ds(0, _CH)], s2)
        d0.wait()
        d3 = pltpu.async_copy(h_hbm.at[src_v], rows_v, s3)
        d1.wait()
        d2.wait()
        d3.wait()
        lo = jnp.maximum(e0 - base, 0)
        hi = jnp.minimum(e1 - base, _CH)

        def edge_body(e, _):
            d = dst_v[pl.ds(e, 16)][0] - nbase
            nm = nrm_v[pl.ds(e, 16)][0]
            for cc in range(grp):
                off = d * w + cc * 16
                acc_v[pl.ds(off, 16)] = (acc_v[pl.ds(off, 16)]
                                         + rows_v[e, pl.ds(cc * 16, 16)] * nm)
            return 0

        lax.fori_loop(lo, hi, edge_body, 0, unroll=False)
        return 0

    lax.fori_loop(0, nch, chunk_body, 0, unroll=False)
    pltpu.sync_copy(acc_v, o_hbm.at[pl.ds(nbase * w, _NPW * w)])


@functools.lru_cache(maxsize=None)
def _make_spmm(w):
    mesh = plsc.VectorSubcoreMesh(core_axis_name="c", subcore_axis_name="s")
    return pl.kernel(
        functools.partial(_spmm_body, w),
        out_type=jax.ShapeDtypeStruct((_NNP * w,), jnp.float32),
        mesh=mesh,
        scratch_types=[
            pltpu.VMEM((48,), jnp.int32),
            pltpu.VMEM((_CH,), jnp.int32),
            pltpu.VMEM((_CH + 16,), jnp.int32),
            pltpu.VMEM((_CH + 16,), jnp.float32),
            pltpu.VMEM((_CH, 128), jnp.float32),
            pltpu.VMEM((_NPW * w,), jnp.float32),
            pltpu.SemaphoreType.DMA,
            pltpu.SemaphoreType.DMA,
            pltpu.SemaphoreType.DMA,
            pltpu.SemaphoreType.DMA,
        ],
    )


def _spmm_call(h, srcs, dsts, nrms, rs, zeros, w):
    return _make_spmm(w)(h, srcs, dsts, nrms, rs, zeros).reshape(_NNP, w)


def _mm_plain(x, wmat):
    m, k = x.shape
    n = wmat.shape[1]

    def body(x_ref, w_ref, o_ref):
        o_ref[...] = jnp.dot(x_ref[...], w_ref[...],
                             preferred_element_type=jnp.float32)

    return pl.pallas_call(
        body,
        grid=(_GRID_M,),
        in_specs=[pl.BlockSpec((_BM, k), lambda i: (i, 0)),
                  pl.BlockSpec((k, n), lambda i: (0, 0))],
        out_specs=pl.BlockSpec((_BM, n), lambda i: (i, 0)),
        out_shape=jax.ShapeDtypeStruct((m, n), jnp.float32),
    )(x, wmat)


def _mm_fused(x, avec, cvec, wmat):
    m, k = x.shape
    n = wmat.shape[1]

    def body(x_ref, a_ref, c_ref, w_ref, o_ref):
        z = jnp.maximum(x_ref[...] * a_ref[...] + c_ref[...], 0.0)
        o_ref[...] = jnp.dot(z, w_ref[...], preferred_element_type=jnp.float32)

    return pl.pallas_call(
        body,
        grid=(_GRID_M,),
        in_specs=[pl.BlockSpec((_BM, k), lambda i: (i, 0)),
                  pl.BlockSpec((1, k), lambda i: (0, 0)),
                  pl.BlockSpec((1, k), lambda i: (0, 0)),
                  pl.BlockSpec((k, n), lambda i: (0, 0))],
        out_specs=pl.BlockSpec((_BM, n), lambda i: (i, 0)),
        out_shape=jax.ShapeDtypeStruct((m, n), jnp.float32),
    )(x, avec, cvec, wmat)


def _elem_diag_bias(s1, t, dis2, bvec):
    m, n = s1.shape

    def body(s_ref, t_ref, d_ref, b_ref, o_ref):
        r = s_ref[...] + t_ref[...][:, 0:n] * d_ref[...] + b_ref[...]
        o_ref[...] = jnp.concatenate(
            [r, jnp.zeros((_BM, 128 - n), jnp.float32)], axis=1)

    return pl.pallas_call(
        body,
        grid=(_GRID_M,),
        in_specs=[pl.BlockSpec((_BM, n), lambda i: (i, 0)),
                  pl.BlockSpec((_BM, 128), lambda i: (i, 0)),
                  pl.BlockSpec((_BM, 1), lambda i: (i, 0)),
                  pl.BlockSpec((1, n), lambda i: (0, 0))],
        out_specs=pl.BlockSpec((_BM, 128), lambda i: (i, 0)),
        out_shape=jax.ShapeDtypeStruct((m, 128), jnp.float32),
    )(s1, t, dis2, bvec)


def _mm_bias_stats(s2, h1, dis2, wmat, bvec):
    m, k = s2.shape
    n = wmat.shape[1]

    def body(s_ref, h_ref, d_ref, w_ref, b_ref, o_ref, sum_ref, sq_ref):
        i = pl.program_id(0)
        u = s_ref[...] + h_ref[...][:, 0:k] * d_ref[...]
        h2 = jnp.dot(u, w_ref[...], preferred_element_type=jnp.float32) \
            + b_ref[...]
        o_ref[...] = h2
        rows = i * _BM + lax.broadcasted_iota(jnp.int32, (_BM, 1), 0)
        h2m = jnp.where(rows < _NN, h2, 0.0)
        s = jnp.sum(h2m, axis=0, keepdims=True)
        q = jnp.sum(h2m * h2m, axis=0, keepdims=True)

        @pl.when(i == 0)
        def _():
            sum_ref[...] = jnp.zeros_like(sum_ref)
            sq_ref[...] = jnp.zeros_like(sq_ref)

        sum_ref[...] += jnp.broadcast_to(s, (8, n))
        sq_ref[...] += jnp.broadcast_to(q, (8, n))

    return pl.pallas_call(
        body,
        grid=(_GRID_M,),
        in_specs=[pl.BlockSpec((_BM, k), lambda i: (i, 0)),
                  pl.BlockSpec((_BM, 128), lambda i: (i, 0)),
                  pl.BlockSpec((_BM, 1), lambda i: (i, 0)),
                  pl.BlockSpec((k, n), lambda i: (0, 0)),
                  pl.BlockSpec((1, n), lambda i: (0, 0))],
        out_specs=[pl.BlockSpec((_BM, n), lambda i: (i, 0)),
                   pl.BlockSpec((8, n), lambda i: (0, 0)),
                   pl.BlockSpec((8, n), lambda i: (0, 0))],
        out_shape=[jax.ShapeDtypeStruct((m, n), jnp.float32),
                   jax.ShapeDtypeStruct((8, n), jnp.float32),
                   jax.ShapeDtypeStruct((8, n), jnp.float32)],
    )(s2, h1, dis2, wmat, bvec)


def _pooling(h, avec, cvec, batch3d):
    m, n = h.shape

    def body(h_ref, a_ref, c_ref, b_ref, o_ref):
        i = pl.program_id(0)
        x = jnp.maximum(h_ref[...] * a_ref[...] + c_ref[...], 0.0)
        ones = jnp.ones((_BM, 1), jnp.float32)
        xe = jnp.concatenate([x, ones], axis=1)
        bt = b_ref[0]
        gids = lax.broadcasted_iota(jnp.int32, (_NG, _BM), 0)
        oh = jnp.where(gids == jnp.broadcast_to(bt, (_NG, _BM)), 1.0, 0.0)
        contrib = jax.lax.dot_general(oh, xe, (((1,), (0,)), ((), ())),
                                      preferred_element_type=jnp.float32)

        @pl.when(i == 0)
        def _():
            o_ref[...] = jnp.zeros_like(o_ref)

        o_ref[...] += contrib

    return pl.pallas_call(
        body,
        grid=(_GRID_M,),
        in_specs=[pl.BlockSpec((_BM, n), lambda i: (i, 0)),
                  pl.BlockSpec((1, n), lambda i: (0, 0)),
                  pl.BlockSpec((1, n), lambda i: (0, 0)),
                  pl.BlockSpec((1, 1, _BM), lambda i: (i, 0, 0))],
        out_specs=pl.BlockSpec((_NG, n + 1), lambda i: (0, 0)),
        out_shape=jax.ShapeDtypeStruct((_NG, n + 1), jnp.float32),
    )(h, avec, cvec, batch3d)


def _protein(target, emb, wlist, blist):
    bn = 8
    structure = [[0], [1, 2], [3, 4, 5]]

    def body(t_ref, e_ref, w0, w1, w2, w3, w4, w5, b0, b1, b2, b3, b4, b5,
             o_ref):
        wrefs = [w0, w1, w2, w3, w4, w5]
        brefs = [b0, b1, b2, b3, b4, b5]
        tgt = t_ref[...]
        oh = jnp.where(tgt ==
                       lax.broadcasted_iota(jnp.int32, (bn * 512, 32), 1),
                       1.0, 0.0)
        e = jnp.dot(oh, e_ref[...], preferred_element_type=jnp.float32)
        rows = []
        for nn in range(bn):
            seq = e[nn * 512:(nn + 1) * 512, :]
            feats = []
            for convs in structure:
                cur = seq
                ln = 512
                for ci in convs:
                    wr = wrefs[ci]
                    acc = jnp.dot(cur[0:ln - 2, :], wr[0],
                                  preferred_element_type=jnp.float32)
                    acc += jnp.dot(cur[1:ln - 1, :], wr[1],
                                   preferred_element_type=jnp.float32)
                    acc += jnp.dot(cur[2:ln, :], wr[2],
                                   preferred_element_type=jnp.float32)
                    cur = jnp.maximum(acc + brefs[ci][...], 0.0)
                    ln -= 2
                feats.append(jnp.max(cur, axis=0, keepdims=True))
            rows.append(jnp.concatenate(feats, axis=1))
        o_ref[...] = jnp.concatenate(rows, axis=0)

    in_specs = [pl.BlockSpec((bn * 512, 1), lambda i: (i, 0)),
                pl.BlockSpec((32, 128), lambda i: (0, 0))]
    for wshape in [(3, 128, 96), (3, 128, 96), (3, 96, 96),
                   (3, 128, 96), (3, 96, 96), (3, 96, 96)]:
        in_specs.append(pl.BlockSpec(wshape, lambda i: (0, 0, 0)))
    for _ in range(6):
        in_specs.append(pl.BlockSpec((1, 96), lambda i: (0, 0)))

    return pl.pallas_call(
        body,
        grid=(_NG // bn,),
        in_specs=in_specs,
        out_specs=pl.BlockSpec((bn, 288), lambda i: (i, 0)),
        out_shape=jax.ShapeDtypeStruct((_NG, 288), jnp.float32),
    )(target, emb, *wlist, *blist)


def _head(sums_counts, pcat, ligw, ligb, protw, protb, cws, cbs):
    def body(sc_ref, pc_ref, lw, lb, pw, pb, w1, b1, w2, b2, w3, b3, w4, b4,
             o_ref):
        sums = sc_ref[:, 0:228]
        counts = sc_ref[:, 228:229]
        pooled = sums / jnp.maximum(counts, 1.0)
        ligand = jnp.dot(pooled, lw[...],
                         preferred_element_type=jnp.float32) + lb[...]
        protein = jnp.dot(pc_ref[...], pw[...],
                          preferred_element_type=jnp.float32) + pb[...]
        z = jnp.concatenate([protein, ligand], axis=1)
        z = jnp.maximum(jnp.dot(z, w1[...],
                                preferred_element_type=jnp.float32) + b1[...],
                        0.0)
        z = jnp.maximum(jnp.dot(z, w2[...],
                                preferred_element_type=jnp.float32) + b2[...],
                        0.0)
        z = jnp.maximum(jnp.dot(z, w3[...],
                                preferred_element_type=jnp.float32) + b3[...],
                        0.0)
        o_ref[...] = jnp.dot(z, w4[...],
                             preferred_element_type=jnp.float32) + b4[...]

    args = [sums_counts, pcat, ligw, ligb, protw, protb,
            cws[0], cbs[0], cws[1], cbs[1], cws[2], cbs[2], cws[3], cbs[3]]
    return pl.pallas_call(
        body,
        out_shape=jax.ShapeDtypeStruct((_NG, 128), jnp.float32),
    )(*args)


def _stats_to_affine(ssum, ssq, gamma, beta):
    mean = ssum[0] / _NN
    var = ssq[0] / _NN - mean * mean
    a = gamma * lax.rsqrt(var + 1e-5)
    c = beta - mean * a
    return a, c


def _pad_cols(wmat):
    return jnp.pad(wmat, ((0, 0), (0, 128 - wmat.shape[1])))


def _gcb_tail(t, p, scf, dis2col):
    w = p['b1'].shape[0]
    s1 = _spmm_call(t, *scf(w), w)
    h1 = _elem_diag_bias(s1, t, dis2col, p['b1'][None, :])
    s2 = _spmm_call(h1, *scf(w), w)
    h2, ssum, ssq = _mm_bias_stats(s2, h1, dis2col, p['W2'], p['b2'][None, :])
    a, c = _stats_to_affine(ssum, ssq, p['gamma'], p['beta'])
    return h2, a, c


def kernel(x, edge_index, batch, target, params):
    f32 = jnp.float32
    src = edge_index[0]
    dst = edge_index[1]

    deg = jnp.zeros((_NN,), f32).at[dst].add(1.0) + 1.0
    dis = lax.rsqrt(deg)
    order = jnp.argsort(dst)
    srcs = src[order]
    dsts = dst[order]
    nrms = dis[srcs] * dis[dsts]
    srcs = jnp.concatenate([srcs, jnp.zeros((_CH,), jnp.int32)])
    dsts_p = jnp.concatenate([dsts, jnp.zeros((_CH,), jnp.int32)])
    nrms = jnp.concatenate([nrms, jnp.zeros((_CH,), f32)])
    bounds = jnp.arange(0, _NNP + 1, _NPW, dtype=jnp.int32)
    rs = jnp.searchsorted(dsts, bounds).astype(jnp.int32)
    rs = jnp.concatenate([rs, jnp.full((48 - 33,), _NE, jnp.int32)])
    dis2 = jnp.pad(dis * dis, (0, _NNP - _NN)).reshape(_NNP, 1)
    z16 = jnp.zeros((_NPW * 16,), f32)
    z32 = jnp.zeros((_NPW * 32,), f32)

    def sc_args(w):
        return (srcs, dsts_p, nrms, rs, z16 if w == 16 else z32)

    lig = params['ligand']
    xp = jnp.pad(x, ((0, _NNP - _NN), (0, 128 - x.shape[1])))

    p = lig['convn']
    t = _mm_plain(xp, _pad_cols(jnp.pad(p['W1'], ((0, 128 - 78), (0, 0)))))
    h, a, c = _gcb_tail(t, p, sc_args, dis2)

    for layers, trans in zip(lig['blocks'], lig['transitions']):
        win = h.shape[1]
        bw = win + 8 * 32
        cat = jnp.zeros((_NNP, bw), f32)
        cat = lax.dynamic_update_slice(cat, h, (0, 0))
        alist = [a]
        clist = [c]
        widths = [win]
        for lp in layers:
            cin = sum(widths)
            ca = jnp.concatenate(alist + [jnp.ones((bw - cin,), f32)])
            cc = jnp.concatenate(clist + [jnp.zeros((bw - cin,), f32)])
            w1p = jnp.pad(lp['in']['W1'], ((0, bw - cin), (0, 0)))
            t = _mm_fused(cat, ca[None, :], cc[None, :], _pad_cols(w1p))
            t2, a2, c2 = _gcb_tail(t, lp['in'], sc_args, dis2)
            u = _mm_fused(t2, a2[None, :], c2[None, :],
                          _pad_cols(lp['out']['W1']))
            hout, ao, co = _gcb_tail(u, lp['out'], sc_args, dis2)
            cat = lax.dynamic_update_slice(cat, hout, (0, cin))
            alist.append(ao)
            clist.append(co)
            widths.append(32)
        ca = jnp.concatenate(alist)
        cc = jnp.concatenate(clist)
        t = _mm_fused(cat, ca[None, :], cc[None, :], _pad_cols(trans['W1']))
        h, a, c = _gcb_tail(t, trans, sc_args, dis2)

    batch3d = jnp.pad(batch, (0, _NNP - _NN), constant_values=_NG) \
        .reshape(_GRID_M, 1, _BM)
    sums_counts = _pooling(h, a[None, :], c[None, :], batch3d)

    prot = params['protein']
    wlist = []
    blist = []
    for convs in prot['blocks']:
        for cparam in convs:
            wlist.append(jnp.transpose(cparam['W'], (2, 1, 0)))
            blist.append(cparam['b'][None, :])
    embp = jnp.pad(prot['embed'], ((0, 32 - 26), (0, 0)))
    pcat = _protein(target.reshape(_NG * 512, 1), embp, wlist, blist)

    cls = params['classifier']
    cws = [cl['W'] for cl in cls]
    cbs = [cl['b'][None, :] for cl in cls]
    cws[3] = jnp.pad(cws[3], ((0, 0), (0, 127)))
    cbs[3] = jnp.pad(cbs[3], ((0, 0), (0, 127)))
    out = _head(sums_counts, pcat,
                lig['cls']['W'], lig['cls']['b'][None, :],
                prot['linear']['W'], prot['linear']['b'][None, :],
                cws, cbs)
    return out[:, :1]
